# baseline (speedup 1.0000x reference)
import jax
import jax.numpy as jnp
from jax.experimental import pallas as pl

_B = 32
_D = 64
_N = 32
_X_MAX = 100.0
_ALPHA = 0.75


def _glove_kernel(cidx_ref, uidx_ref, v_ref, u_ref, vb_ref, ub_ref, co_ref, out_ref):
    c = cidx_ref[0, :]
    u = uidx_ref[0, :]
    col = jax.lax.broadcasted_iota(jnp.int32, (_B, _N), 1)
    onehot_c = (col == c[:, None]).astype(jnp.float32)
    onehot_u = (col == u[:, None]).astype(jnp.float32)

    V = v_ref[...]
    U = u_ref[...]
    E = jnp.dot(V, U.T, preferred_element_type=jnp.float32)
    selC = jnp.dot(onehot_c, E, preferred_element_type=jnp.float32)
    dots = jnp.sum(selC * onehot_u, axis=1, keepdims=True)

    cb = jnp.dot(onehot_c, vb_ref[...], preferred_element_type=jnp.float32)
    tb = jnp.dot(onehot_u, ub_ref[...], preferred_element_type=jnp.float32)

    selCo = jnp.dot(onehot_c, co_ref[...], preferred_element_type=jnp.float32)
    co = jnp.sum(selCo * onehot_u, axis=1, keepdims=True)

    w = jnp.where(co < _X_MAX, (co / _X_MAX) ** _ALPHA, 1.0)
    resid = dots + cb + tb - jnp.log(co)
    out_ref[...] = jnp.sum(resid * resid * w, keepdims=True)


def kernel(center_word_lookup, context_word_lookup, emb_V, emb_U, v_bias, u_bias, comat):
    cidx = center_word_lookup.astype(jnp.int32).reshape(1, _B)
    uidx = context_word_lookup.astype(jnp.int32).reshape(1, _B)
    out = pl.pallas_call(
        _glove_kernel,
        out_shape=jax.ShapeDtypeStruct((1, 1), jnp.float32),
    )(cidx, uidx, emb_V[:_N], emb_U[:_N], v_bias[:_N], u_bias[:_N], comat)
    return out[0, 0]
